# baseline (speedup 1.0000x reference)
import jax
import jax.numpy as jnp
import numpy as np
from jax import lax
from jax.experimental import pallas as pl
from jax.experimental.pallas import tpu as pltpu
from jax.experimental.pallas import tpu_sc as plsc

N = 10000
E = 320000
D = 128
DH = D // 2

NC = 2
NS = 16
NW = NC * NS

K = 125
EPT = E // NS
CH = EPT // K
NBUF = 4
KP = 128
RPT = 624
TAIL = N - RPT * NS
ZR = RPT // 8
OR = RPT // 6
LANES = DH // 16

_PERM = np.empty((D,), np.int32)
for _c in range(NC):
    for _j in range(DH // 32):
        for _i in range(16):
            for _h in range(2):
                _PERM[_c * DH + 32 * _j + 2 * _i + _h] = (
                    _c * DH + 32 * _j + 16 * _h + _i)


def _spmm_body(g_hbm, e_hbm, out_hbm,
               acc, gbuf_a, sbuf_a, src_v, dst_v, wring,
               gsem_a, ssem_a, wsem_a, zsem):
    c = lax.axis_index("c")
    s = lax.axis_index("s")
    row0 = s * CH

    gbuf = tuple(gbuf_a.at[b] for b in range(NBUF))
    sbuf = tuple(sbuf_a.at[b] for b in range(NBUF))
    sbuf0 = sbuf[0]
    gsem = tuple(gsem_a.at[b] for b in range(NBUF))
    ssem = tuple(ssem_a.at[b] for b in range(NBUF))
    wsem = tuple(wsem_a.at[b] for b in range(NBUF))

    pltpu.sync_copy(e_hbm.at[1, pl.ds(row0, CH)], src_v)
    pltpu.sync_copy(e_hbm.at[0, pl.ds(row0, CH)], dst_v)

    @plsc.parallel_loop(0, K, step=1, unroll=5)
    def _zero_row(r):
        for j in range(LANES):
            sbuf0[r, pl.ds(j * 16, 16)] = jnp.zeros((16,), jnp.float32)
    for i in range(RPT // ZR):
        pltpu.async_copy(sbuf0.at[pl.ds(0, ZR)],
                         acc.at[pl.ds(s * RPT + i * ZR, ZR)], zsem)

    @pl.when(s == NS - 1)
    def _():
        pltpu.async_copy(sbuf0.at[pl.ds(0, TAIL)],
                         acc.at[pl.ds(RPT * NS, TAIL)], zsem)

    def _gather(b, g):
        pltpu.async_copy(g_hbm.at[c].at[src_v.at[g]], gbuf[b], gsem[b])

    def _gather_wait(b, g):
        pltpu.make_async_copy(g_hbm.at[c].at[src_v.at[g]],
                              gbuf[b], gsem[b]).wait()

    def _wfetch(b, g):
        pltpu.async_copy(e_hbm.at[2, row0 + g],
                         wring.at[pl.ds(b * KP, K)], wsem[b])

    def _wfetch_wait(b, g):
        pltpu.make_async_copy(e_hbm.at[2, row0 + g],
                              wring.at[pl.ds(b * KP, K)], wsem[b]).wait()

    def _scatter(b, g):
        pltpu.async_copy(sbuf[b], acc.at[dst_v.at[g]], ssem[b], add=True)

    def _scatter_wait(b, g):
        pltpu.make_async_copy(sbuf[b], acc.at[dst_v.at[g]], ssem[b]).wait()

    def _scale(b):
        gm = gbuf[b]
        sm = sbuf[b]
        base = b * KP

        @plsc.parallel_loop(0, K, step=1, unroll=5)
        def body(k):
            ki = jnp.full((16,), base + k, jnp.int32)
            wk = plsc.bitcast(plsc.load_gather(wring, [ki]),
                              jnp.float32)
            for j in range(DH // 32):
                v = gm[k, pl.ds(j * 32, 32)]
                lo, hi = plsc.unpack(v, format=plsc.PackFormat.INTERLEAVED)
                sm[k, pl.ds(j * 32, 16)] = lo * wk
                sm[k, pl.ds(j * 32 + 16, 16)] = hi * wk

    for b in range(1, NBUF):
        _wfetch(b, b)
        _gather(b, b)

    for i in range(RPT // ZR):
        pltpu.make_async_copy(sbuf0.at[pl.ds(0, ZR)],
                              acc.at[pl.ds(s * RPT + i * ZR, ZR)],
                              zsem).wait()

    @pl.when(s == NS - 1)
    def _():
        pltpu.make_async_copy(sbuf0.at[pl.ds(0, TAIL)],
                              acc.at[pl.ds(RPT * NS, TAIL)], zsem).wait()

    _wfetch(0, 0)
    _gather(0, 0)

    plsc.subcore_barrier()

    def _chunk_quad(gq, carry):
        for b in range(NBUF):
            g = NBUF * gq + b
            _gather_wait(b, g)
            _wfetch_wait(b, g)

            @pl.when(g >= NBUF)
            def _():
                _scatter_wait(b, g - NBUF)

            _scale(b)
            _scatter(b, g)

            @pl.when(g + NBUF < CH)
            def _():
                _wfetch(b, g + NBUF)
                _gather(b, g + NBUF)
        return carry

    lax.fori_loop(0, CH // NBUF, _chunk_quad, None)
    for b in range(NBUF):
        _scatter_wait(b, CH - NBUF + b)

    plsc.subcore_barrier()

    NOUT = RPT // OR

    def _oload(i):
        pltpu.async_copy(acc.at[pl.ds(s * RPT + i * OR, OR)],
                         sbuf[i % NBUF].at[pl.ds(0, OR)], gsem[i % NBUF])

    def _oload_wait(i):
        pltpu.make_async_copy(acc.at[pl.ds(s * RPT + i * OR, OR)],
                              sbuf[i % NBUF].at[pl.ds(0, OR)],
                              gsem[i % NBUF]).wait()

    def _ostore(i):
        pltpu.async_copy(sbuf[i % NBUF].at[pl.ds(0, OR)],
                         out_hbm.at[pl.ds(s * RPT + i * OR, OR),
                                    pl.ds(c * DH, DH)], ssem[i % NBUF])

    def _ostore_wait(i):
        pltpu.make_async_copy(sbuf[i % NBUF].at[pl.ds(0, OR)],
                              out_hbm.at[pl.ds(s * RPT + i * OR, OR),
                                         pl.ds(c * DH, DH)],
                              ssem[i % NBUF]).wait()

    def _relu_rows(buf, nrows):
        @plsc.parallel_loop(0, nrows, step=1, unroll=5)
        def _r(r):
            for j in range(LANES):
                sl = pl.ds(j * 16, 16)
                buf[r, sl] = jnp.maximum(buf[r, sl], 0.0)

    _oload(0)
    _oload(1)
    for i in range(NOUT):
        _oload_wait(i)
        _relu_rows(sbuf[i % NBUF], OR)
        _ostore(i)
        if i + 2 < NOUT:
            if i >= 2:
                _ostore_wait(i - 2)
            _oload(i + 2)
    for i in range(max(0, NOUT - NBUF), NOUT):
        _ostore_wait(i)

    @pl.when(s == NS - 1)
    def _():
        r0 = RPT * NS
        pltpu.sync_copy(acc.at[pl.ds(r0, TAIL)], sbuf0.at[pl.ds(0, TAIL)])
        _relu_rows(sbuf0, TAIL)
        pltpu.sync_copy(sbuf0.at[pl.ds(0, TAIL)],
                        out_hbm.at[pl.ds(r0, TAIL), pl.ds(c * DH, DH)])


@jax.jit
def _spmm(g2, edges):
    mesh = plsc.VectorSubcoreMesh(core_axis_name="c", subcore_axis_name="s",
                                  num_cores=NC, num_subcores=NS)
    return pl.kernel(
        _spmm_body,
        out_type=jax.ShapeDtypeStruct((N, D), jnp.float32),
        mesh=mesh,
        compiler_params=pltpu.CompilerParams(needs_layout_passes=False,
                                             use_tc_tiling_on_sc=False),
        scratch_types=[
            pltpu.VMEM_SHARED((N, DH), jnp.float32),
            pltpu.VMEM((NBUF, K, DH), jnp.bfloat16),
            pltpu.VMEM((NBUF, K, DH), jnp.float32),
            pltpu.VMEM((CH, K), jnp.int32),
            pltpu.VMEM((CH, K), jnp.int32),
            pltpu.VMEM((NBUF * KP,), jnp.int32),
            pltpu.SemaphoreType.DMA((NBUF,)),
            pltpu.SemaphoreType.DMA((NBUF,)),
            pltpu.SemaphoreType.DMA((NBUF,)),
            pltpu.SemaphoreType.DMA,
        ],
    )(g2, edges)


def _transform_body(h_ref, w_ref, g_ref):
    h = h_ref[...]
    for i in range(NC):
        g_ref[i] = lax.dot_general(
            h, w_ref[pl.ds(i * DH, DH), :],
            dimension_numbers=(((1,), (1,)), ((), ())),
            preferred_element_type=jnp.float32).astype(jnp.bfloat16)


@jax.jit
def _transform(h, w):
    return pl.pallas_call(
        _transform_body,
        out_shape=jax.ShapeDtypeStruct((NC, N, DH), jnp.bfloat16),
    )(h, w)


def kernel(H, edge_index, edge_weight, W):
    edges = jnp.concatenate(
        [edge_index, lax.bitcast_convert_type(edge_weight, jnp.int32)[None]],
        axis=0).reshape(3, NS * CH, K)
    g2 = _transform(H[...], W[jnp.asarray(_PERM)])
    return _spmm(g2, edges)


# speedup vs baseline: 1.0055x; 1.0055x over previous
import jax
import jax.numpy as jnp
import numpy as np
from jax import lax
from jax.experimental import pallas as pl
from jax.experimental.pallas import tpu as pltpu
from jax.experimental.pallas import tpu_sc as plsc

N = 10000
E = 320000
D = 128
DH = D // 2

NC = 2
NS = 16
NW = NC * NS

K = 125
EPT = E // NS
CH = EPT // K
NBUF = 4
KP = 128
RPT = 624
TAIL = N - RPT * NS
ZR = RPT // 8
OR = RPT // 6
LANES = DH // 16

_PERM = np.empty((D,), np.int32)
for _c in range(NC):
    for _j in range(DH // 32):
        for _i in range(16):
            for _h in range(2):
                _PERM[_c * DH + 32 * _j + 2 * _i + _h] = (
                    _c * DH + 32 * _j + 16 * _h + _i)


def _spmm_body(g_hbm, e_hbm, out_hbm,
               acc, gbuf_a, sbuf_a, src_v, dring, wring,
               gsem_a, ssem_a, dsem_a, wsem_a, zsem):
    c = lax.axis_index("c")
    s = lax.axis_index("s")
    row0 = s * CH

    gbuf = tuple(gbuf_a.at[b] for b in range(NBUF))
    sbuf = tuple(sbuf_a.at[b] for b in range(NBUF))
    sbuf0 = sbuf[0]
    gsem = tuple(gsem_a.at[b] for b in range(NBUF))
    ssem = tuple(ssem_a.at[b] for b in range(NBUF))
    dsem = tuple(dsem_a.at[b] for b in range(NBUF))
    wsem = tuple(wsem_a.at[b] for b in range(NBUF))

    pltpu.sync_copy(e_hbm.at[1, pl.ds(row0, CH)], src_v)

    @plsc.parallel_loop(0, K, step=1, unroll=5)
    def _zero_row(r):
        for j in range(LANES):
            sbuf0[r, pl.ds(j * 16, 16)] = jnp.zeros((16,), jnp.float32)
    for i in range(RPT // ZR):
        pltpu.async_copy(sbuf0.at[pl.ds(0, ZR)],
                         acc.at[pl.ds(s * RPT + i * ZR, ZR)], zsem)

    @pl.when(s == NS - 1)
    def _():
        pltpu.async_copy(sbuf0.at[pl.ds(0, TAIL)],
                         acc.at[pl.ds(RPT * NS, TAIL)], zsem)

    def _gather(b, g):
        pltpu.async_copy(g_hbm.at[c].at[src_v.at[g]], gbuf[b], gsem[b])

    def _gather_wait(b, g):
        pltpu.make_async_copy(g_hbm.at[c].at[src_v.at[g]],
                              gbuf[b], gsem[b]).wait()

    def _dfetch(b, g, dslot):
        pltpu.async_copy(e_hbm.at[0, row0 + g], dring.at[dslot], dsem[b])

    def _dfetch_wait(b, g, dslot):
        pltpu.make_async_copy(e_hbm.at[0, row0 + g],
                              dring.at[dslot], dsem[b]).wait()

    def _wfetch(b, g):
        pltpu.async_copy(e_hbm.at[2, row0 + g],
                         wring.at[pl.ds(b * KP, K)], wsem[b])

    def _wfetch_wait(b, g):
        pltpu.make_async_copy(e_hbm.at[2, row0 + g],
                              wring.at[pl.ds(b * KP, K)], wsem[b]).wait()

    def _scatter(b, g, dslot):
        pltpu.async_copy(sbuf[b], acc.at[dring.at[dslot]], ssem[b], add=True)

    def _scatter_wait(b, g, dslot):
        pltpu.make_async_copy(sbuf[b], acc.at[dring.at[dslot]],
                              ssem[b]).wait()

    def _scale(b):
        gm = gbuf[b]
        sm = sbuf[b]
        base = b * KP

        @plsc.parallel_loop(0, K, step=1, unroll=5)
        def body(k):
            ki = jnp.full((16,), base + k, jnp.int32)
            wk = plsc.bitcast(plsc.load_gather(wring, [ki]),
                              jnp.float32)
            for j in range(DH // 32):
                v = gm[k, pl.ds(j * 32, 32)]
                lo, hi = plsc.unpack(v, format=plsc.PackFormat.INTERLEAVED)
                sm[k, pl.ds(j * 32, 16)] = lo * wk
                sm[k, pl.ds(j * 32 + 16, 16)] = hi * wk

    for b in range(1, NBUF):
        _wfetch(b, b)
        _dfetch(b, b, b)
        _gather(b, b)

    for i in range(RPT // ZR):
        pltpu.make_async_copy(sbuf0.at[pl.ds(0, ZR)],
                              acc.at[pl.ds(s * RPT + i * ZR, ZR)],
                              zsem).wait()

    @pl.when(s == NS - 1)
    def _():
        pltpu.make_async_copy(sbuf0.at[pl.ds(0, TAIL)],
                              acc.at[pl.ds(RPT * NS, TAIL)], zsem).wait()

    _wfetch(0, 0)
    _dfetch(0, 0, 0)
    _gather(0, 0)

    plsc.subcore_barrier()

    def _chunk_oct(gq2, carry):
        for q in range(2):
            for b in range(NBUF):
                g = 2 * NBUF * gq2 + q * NBUF + b
                dslot = q * NBUF + b
                _gather_wait(b, g)
                _wfetch_wait(b, g)

                @pl.when(g >= NBUF)
                def _():
                    _scatter_wait(b, g - NBUF, (dslot + NBUF) % (2 * NBUF))

                _scale(b)
                _dfetch_wait(b, g, dslot)
                _scatter(b, g, dslot)

                @pl.when(g + NBUF < CH)
                def _():
                    _wfetch(b, g + NBUF)
                    _dfetch(b, g + NBUF, (dslot + NBUF) % (2 * NBUF))
                    _gather(b, g + NBUF)
        return carry

    lax.fori_loop(0, CH // (2 * NBUF), _chunk_oct, None)
    for b in range(NBUF):
        _scatter_wait(b, CH - NBUF + b, (CH - NBUF + b) % (2 * NBUF))

    plsc.subcore_barrier()

    NOUT = RPT // OR

    def _oload(i):
        pltpu.async_copy(acc.at[pl.ds(s * RPT + i * OR, OR)],
                         sbuf[i % NBUF].at[pl.ds(0, OR)], gsem[i % NBUF])

    def _oload_wait(i):
        pltpu.make_async_copy(acc.at[pl.ds(s * RPT + i * OR, OR)],
                              sbuf[i % NBUF].at[pl.ds(0, OR)],
                              gsem[i % NBUF]).wait()

    def _ostore(i):
        pltpu.async_copy(sbuf[i % NBUF].at[pl.ds(0, OR)],
                         out_hbm.at[pl.ds(s * RPT + i * OR, OR),
                                    pl.ds(c * DH, DH)], ssem[i % NBUF])

    def _ostore_wait(i):
        pltpu.make_async_copy(sbuf[i % NBUF].at[pl.ds(0, OR)],
                              out_hbm.at[pl.ds(s * RPT + i * OR, OR),
                                         pl.ds(c * DH, DH)],
                              ssem[i % NBUF]).wait()

    def _relu_rows(buf, nrows):
        @plsc.parallel_loop(0, nrows, step=1, unroll=5)
        def _r(r):
            for j in range(LANES):
                sl = pl.ds(j * 16, 16)
                buf[r, sl] = jnp.maximum(buf[r, sl], 0.0)

    _oload(0)
    _oload(1)
    for i in range(NOUT):
        _oload_wait(i)
        _relu_rows(sbuf[i % NBUF], OR)
        _ostore(i)
        if i + 2 < NOUT:
            if i >= 2:
                _ostore_wait(i - 2)
            _oload(i + 2)
    for i in range(max(0, NOUT - NBUF), NOUT):
        _ostore_wait(i)

    @pl.when(s == NS - 1)
    def _():
        r0 = RPT * NS
        pltpu.sync_copy(acc.at[pl.ds(r0, TAIL)], sbuf0.at[pl.ds(0, TAIL)])
        _relu_rows(sbuf0, TAIL)
        pltpu.sync_copy(sbuf0.at[pl.ds(0, TAIL)],
                        out_hbm.at[pl.ds(r0, TAIL), pl.ds(c * DH, DH)])


@jax.jit
def _spmm(g2, edges):
    mesh = plsc.VectorSubcoreMesh(core_axis_name="c", subcore_axis_name="s",
                                  num_cores=NC, num_subcores=NS)
    return pl.kernel(
        _spmm_body,
        out_type=jax.ShapeDtypeStruct((N, D), jnp.float32),
        mesh=mesh,
        compiler_params=pltpu.CompilerParams(needs_layout_passes=False,
                                             use_tc_tiling_on_sc=False),
        scratch_types=[
            pltpu.VMEM_SHARED((N, DH), jnp.float32),
            pltpu.VMEM((NBUF, K, DH), jnp.bfloat16),
            pltpu.VMEM((NBUF, K, DH), jnp.float32),
            pltpu.VMEM((CH, K), jnp.int32),
            pltpu.VMEM((2 * NBUF, K), jnp.int32),
            pltpu.VMEM((NBUF * KP,), jnp.int32),
            pltpu.SemaphoreType.DMA((NBUF,)),
            pltpu.SemaphoreType.DMA((NBUF,)),
            pltpu.SemaphoreType.DMA((NBUF,)),
            pltpu.SemaphoreType.DMA((NBUF,)),
            pltpu.SemaphoreType.DMA,
        ],
    )(g2, edges)


def _transform_body(h_ref, w_ref, g_ref):
    h = h_ref[...]
    for i in range(NC):
        g_ref[i] = lax.dot_general(
            h, w_ref[pl.ds(i * DH, DH), :],
            dimension_numbers=(((1,), (1,)), ((), ())),
            preferred_element_type=jnp.float32).astype(jnp.bfloat16)


@jax.jit
def _transform(h, w):
    return pl.pallas_call(
        _transform_body,
        out_shape=jax.ShapeDtypeStruct((NC, N, DH), jnp.bfloat16),
    )(h, w)


def kernel(H, edge_index, edge_weight, W):
    edges = jnp.concatenate(
        [edge_index, lax.bitcast_convert_type(edge_weight, jnp.int32)[None]],
        axis=0).reshape(3, NS * CH, K)
    g2 = _transform(H[...], W[jnp.asarray(_PERM)])
    return _spmm(g2, edges)
